# baseline (speedup 1.0000x reference)
import jax
import jax.numpy as jnp
from jax import lax
from jax.experimental import pallas as pl
from jax.experimental.pallas import tpu as pltpu
from jax.experimental.pallas import tpu_sc as plsc

NC = 2
NS = 16
LANES = 16
NW = NC * NS
CHUNK = 128


def _round_up(v, m):
    return (v + m - 1) // m * m


def _sc_degree(cols2d, ones_hbm, zeros_hbm, npad, nch):
    mesh = plsc.VectorSubcoreMesh(core_axis_name="c", subcore_axis_name="s")
    rows_per_sub = npad // NS

    def body(cols_hbm, ones_h, zeros_h, out_hbm, colidx, buf, acc, sem):
        cid = lax.axis_index("c")
        sid = lax.axis_index("s")
        wid = sid * NC + cid
        pltpu.sync_copy(cols_hbm.at[wid], colidx)
        pltpu.sync_copy(ones_h, buf)
        base = sid * rows_per_sub
        pltpu.sync_copy(zeros_h, acc.at[pl.ds(base, rows_per_sub)])
        plsc.subcore_barrier()

        def fire_body(j, carry):
            pltpu.async_copy(buf, acc.at[colidx.at[j]], sem, add=True)
            return carry

        lax.fori_loop(0, nch, fire_body, 0)

        def drain_body(j, carry):
            pltpu.make_async_copy(buf, acc.at[colidx.at[0]], sem).wait()
            return carry

        lax.fori_loop(0, nch, drain_body, 0)
        plsc.subcore_barrier()
        pltpu.sync_copy(acc.at[pl.ds(base, rows_per_sub)],
                        out_hbm.at[cid, pl.ds(base, rows_per_sub)])

    return pl.kernel(
        body,
        out_type=jax.ShapeDtypeStruct((NC, npad, LANES), jnp.float32),
        mesh=mesh,
        compiler_params=pltpu.CompilerParams(use_tc_tiling_on_sc=False),
        scratch_types=[
            pltpu.VMEM((nch, CHUNK), jnp.int32),
            pltpu.VMEM((CHUNK, LANES), jnp.float32),
            pltpu.VMEM_SHARED((npad, LANES), jnp.float32),
            pltpu.SemaphoreType.DMA,
        ],
    )(cols2d, ones_hbm, zeros_hbm)


def _sc_propagate(g_view, rows3d, cols3d, npad, nchp, dh, mult):
    mesh = plsc.VectorSubcoreMesh(core_axis_name="c", subcore_axis_name="s")
    rows_per_sub = npad // NS
    K = 6
    PD = 3
    m_blocks = nchp // K

    def body(g_hbm, rows_hbm, cols_hbm, zeros_hbm, out_hbm,
             rowidx, colidx, *rest):
        rbuf = rest[:K]
        sem_g = rest[K:2 * K]
        sem_s = rest[2 * K:3 * K]
        acc = rest[3 * K]
        cid = lax.axis_index("c")
        sid = lax.axis_index("s")
        pltpu.sync_copy(rows_hbm.at[sid], rowidx)
        pltpu.sync_copy(cols_hbm.at[sid], colidx)
        base = sid * rows_per_sub
        pltpu.sync_copy(zeros_hbm, acc.at[pl.ds(base, rows_per_sub)])

        off = cid + jnp.zeros((LANES,), jnp.int32)
        vpc = CHUNK // LANES

        def shift_body(i, carry):
            r = i // vpc
            sl = pl.ds((i % vpc) * LANES, LANES)
            rowidx[r, sl] = rowidx[r, sl] * mult + off
            return carry

        lax.fori_loop(0, nchp * vpc, shift_body, 0)
        plsc.subcore_barrier()

        def start_gather(u, j):
            pltpu.async_copy(g_hbm.at[rowidx.at[j]], rbuf[u], sem_g[u])

        def wait_gather(u):
            pltpu.make_async_copy(
                g_hbm.at[rowidx.at[0]], rbuf[u], sem_g[u]).wait()

        def start_scatter(u, j):
            pltpu.async_copy(rbuf[u], acc.at[colidx.at[j]], sem_s[u],
                             add=True)

        def wait_scatter(u):
            pltpu.make_async_copy(
                rbuf[u], acc.at[colidx.at[0]], sem_s[u]).wait()

        for u in range(PD):
            start_gather(u, u)

        def visit(u, j, prefetch, fresh_slot):
            wait_gather(u)
            start_scatter(u, j)
            if prefetch:
                up = (u + PD) % K
                if not fresh_slot:
                    wait_scatter(up)
                start_gather(up, j + PD)

        for u in range(K):
            visit(u, u, prefetch=True, fresh_slot=(u < K - PD))

        def mid_block(m, carry):
            j0 = m * K
            for u in range(K):
                visit(u, j0 + u, prefetch=True, fresh_slot=False)
            return carry

        lax.fori_loop(1, m_blocks - 1, mid_block, 0)

        jlast = (m_blocks - 1) * K
        for u in range(K):
            visit(u, jlast + u, prefetch=(u < K - PD), fresh_slot=False)
        for u in range(K):
            wait_scatter(u)
        plsc.subcore_barrier()

        pltpu.sync_copy(acc.at[pl.ds(base, rows_per_sub)],
                        out_hbm.at[pl.ds(base, rows_per_sub),
                                   pl.ds(cid * dh, dh)])

    return pl.kernel(
        body,
        out_type=jax.ShapeDtypeStruct((npad, 128), jnp.float32),
        mesh=mesh,
        compiler_params=pltpu.CompilerParams(use_tc_tiling_on_sc=False),
        scratch_types=(
            [pltpu.VMEM((nchp, CHUNK), jnp.int32)] * 2
            + [pltpu.VMEM((CHUNK, dh), jnp.float32)] * K
            + [pltpu.SemaphoreType.DMA] * (2 * K)
            + [pltpu.VMEM_SHARED((npad, dh), jnp.float32)]
        ),
    )(g_view, rows3d, cols3d,
      jnp.zeros((rows_per_sub, dh), jnp.float32))


def _deg_to_dinv(d_ref, blk, n_valid, i):
    deg = d_ref[0, :, 0:1] + d_ref[1, :, 0:1]
    row = lax.broadcasted_iota(jnp.int32, (blk, 1), 0) + i * blk
    return jnp.where(row < n_valid,
                     lax.rsqrt(jnp.maximum(deg, 1.0)), 0.0)


def _tc1(x, w_fc, b_fc2d, deg128, n_valid, npad, blk=1024):
    _, d_in = x.shape
    d_h = w_fc.shape[0]

    def body(x_ref, w_ref, b_ref, d_ref, g_ref, dinv_ref):
        i = pl.program_id(0)
        dinv = _deg_to_dinv(d_ref, blk, n_valid, i)
        h = lax.dot_general(x_ref[...], w_ref[...],
                            (((1,), (1,)), ((), ())),
                            preferred_element_type=jnp.float32) + b_ref[...]
        g_ref[...] = jnp.where(dinv > 0.0, h * dinv, 0.0)
        dinv_ref[...] = dinv

    return pl.pallas_call(
        body,
        grid=(pl.cdiv(npad, blk),),
        in_specs=[
            pl.BlockSpec((blk, d_in), lambda i: (i, 0)),
            pl.BlockSpec((d_h, d_in), lambda i: (0, 0)),
            pl.BlockSpec((1, d_h), lambda i: (0, 0)),
            pl.BlockSpec((NC, blk, LANES), lambda i: (0, i, 0)),
        ],
        out_specs=[
            pl.BlockSpec((blk, d_h), lambda i: (i, 0)),
            pl.BlockSpec((blk, 1), lambda i: (i, 0)),
        ],
        out_shape=[
            jax.ShapeDtypeStruct((npad, d_h), jnp.float32),
            jax.ShapeDtypeStruct((npad, 1), jnp.float32),
        ],
    )(x, w_fc, b_fc2d, deg128)


def _tc2(s1, dinv2d, w2, d_h, blk=1024):
    npad = s1.shape[0]
    d_out = w2.shape[0]

    def body(s_ref, d_ref, w_ref, g_ref):
        dinv = d_ref[...]
        t = s_ref[:, :d_h] * dinv
        t = 1.0507009873554805 * jnp.where(
            t > 0.0, t, 1.6732632423543772 * (jnp.exp(t) - 1.0))
        h2 = lax.dot_general(t, w_ref[...], (((1,), (1,)), ((), ())),
                             preferred_element_type=jnp.float32)
        g2 = h2 * dinv
        g_ref[...] = jnp.concatenate([g2, jnp.zeros_like(g2)], axis=1)

    return pl.pallas_call(
        body,
        grid=(pl.cdiv(npad, blk),),
        in_specs=[
            pl.BlockSpec((blk, 128), lambda i: (i, 0)),
            pl.BlockSpec((blk, 1), lambda i: (i, 0)),
            pl.BlockSpec((d_out, d_h), lambda i: (0, 0)),
        ],
        out_specs=pl.BlockSpec((blk, 2 * d_out), lambda i: (i, 0)),
        out_shape=jax.ShapeDtypeStruct((npad, 2 * d_out), jnp.float32),
    )(s1, dinv2d, w2)


def _tc3(s2, dinv2d, b2_2d, n, d_out, blk=1000):
    def body(s_ref, d_ref, b_ref, o_ref):
        z = s_ref[:, :d_out] * d_ref[...] + b_ref[...]
        m = jnp.max(z, axis=1, keepdims=True)
        lse = jnp.log(jnp.sum(jnp.exp(z - m), axis=1, keepdims=True))
        o_ref[...] = z - m - lse

    return pl.pallas_call(
        body,
        grid=(n // blk,),
        in_specs=[
            pl.BlockSpec((blk, 128), lambda i: (i, 0)),
            pl.BlockSpec((blk, 1), lambda i: (i, 0)),
            pl.BlockSpec((1, d_out), lambda i: (0, 0)),
        ],
        out_specs=pl.BlockSpec((blk, d_out), lambda i: (i, 0)),
        out_shape=jax.ShapeDtypeStruct((n, d_out), jnp.float32),
    )(s2, dinv2d, b2_2d)


def kernel(x, edge_index, W_fc, b_fc, W2, b2):
    n, d_in = x.shape
    d_h = W_fc.shape[0]
    d_out = W2.shape[0]
    e = edge_index.shape[1]

    npad = _round_up(n + 8, CHUNK)
    total = e + n
    nchp = _round_up(-(-total // (NS * CHUNK)), 6)
    epad = NS * nchp * CHUNK
    nchd = nchp // NC

    loop = jnp.arange(n, dtype=jnp.int32)
    fill = jnp.full((epad - total,), n, jnp.int32)
    rows_flat = jnp.concatenate([edge_index[0], loop, fill])
    cols_flat = jnp.concatenate([edge_index[1], loop, fill])
    rows3d = rows_flat.reshape(NS, nchp, CHUNK)
    cols3d = cols_flat.reshape(NS, nchp, CHUNK)

    deg128 = _sc_degree(
        cols_flat.reshape(NW, nchd, CHUNK),
        jnp.ones((CHUNK, LANES), jnp.float32),
        jnp.zeros((npad // NS, LANES), jnp.float32),
        npad, nchd)
    g1, dinv = _tc1(x, W_fc, b_fc.reshape(1, -1), deg128, n, npad)
    s1 = _sc_propagate(g1.reshape(NC * npad, d_h // NC), rows3d, cols3d,
                       npad, nchp, d_h // NC, NC)
    g2 = _tc2(s1, dinv, W2, d_h)
    s2 = _sc_propagate(g2.reshape(4 * npad, d_out // NC), rows3d, cols3d,
                       npad, nchp, d_out // NC, 4)
    return _tc3(s2, dinv, b2.reshape(1, -1), n, d_out)


# speedup vs baseline: 1.0356x; 1.0356x over previous
import jax
import jax.numpy as jnp
from jax import lax
from jax.experimental import pallas as pl
from jax.experimental.pallas import tpu as pltpu
from jax.experimental.pallas import tpu_sc as plsc

NC = 2
NS = 16
LANES = 16
NW = NC * NS
CHUNK = 128


def _round_up(v, m):
    return (v + m - 1) // m * m


def _sc_degree(cols2d, ones_hbm, zeros_hbm, npad, nch):
    mesh = plsc.VectorSubcoreMesh(core_axis_name="c", subcore_axis_name="s")
    rows_per_sub = npad // NS

    def body(cols_hbm, ones_h, zeros_h, out_hbm, colidx, buf, acc, sem):
        cid = lax.axis_index("c")
        sid = lax.axis_index("s")
        wid = sid * NC + cid
        pltpu.sync_copy(cols_hbm.at[wid], colidx)
        pltpu.sync_copy(ones_h, buf)
        base = sid * rows_per_sub
        pltpu.sync_copy(zeros_h, acc.at[pl.ds(base, rows_per_sub)])
        plsc.subcore_barrier()

        def fire_body(j, carry):
            pltpu.async_copy(buf, acc.at[colidx.at[j]], sem, add=True)
            return carry

        lax.fori_loop(0, nch, fire_body, 0)

        def drain_body(j, carry):
            pltpu.make_async_copy(buf, acc.at[colidx.at[0]], sem).wait()
            return carry

        lax.fori_loop(0, nch, drain_body, 0)
        plsc.subcore_barrier()
        pltpu.sync_copy(acc.at[pl.ds(base, rows_per_sub)],
                        out_hbm.at[cid, pl.ds(base, rows_per_sub)])

    return pl.kernel(
        body,
        out_type=jax.ShapeDtypeStruct((NC, npad, LANES), jnp.float32),
        mesh=mesh,
        compiler_params=pltpu.CompilerParams(use_tc_tiling_on_sc=False),
        scratch_types=[
            pltpu.VMEM((nch, CHUNK), jnp.int32),
            pltpu.VMEM((CHUNK, LANES), jnp.float32),
            pltpu.VMEM_SHARED((npad, LANES), jnp.float32),
            pltpu.SemaphoreType.DMA,
        ],
    )(cols2d, ones_hbm, zeros_hbm)


def _sc_propagate(g_stack, rows3d, cols3d, npad, nchp, dh):
    mesh = plsc.VectorSubcoreMesh(core_axis_name="c", subcore_axis_name="s")
    rows_per_sub = npad // NS
    K = 6
    PD = 3
    m_blocks = nchp // K

    def body(g_hbm, rows_hbm, cols_hbm, zeros_hbm, out_hbm,
             rowidx, colidx, *rest):
        rbuf = rest[:K]
        sem_g = rest[K:2 * K]
        sem_s = rest[2 * K:3 * K]
        acc = rest[3 * K]
        cid = lax.axis_index("c")
        sid = lax.axis_index("s")
        pltpu.sync_copy(rows_hbm.at[sid], rowidx)
        pltpu.sync_copy(cols_hbm.at[sid], colidx)
        base = sid * rows_per_sub
        pltpu.sync_copy(zeros_hbm, acc.at[pl.ds(base, rows_per_sub)])

        off = (cid * npad) + jnp.zeros((LANES,), jnp.int32)
        vpc = CHUNK // LANES

        def shift_body(i, carry):
            r = i // vpc
            sl = pl.ds((i % vpc) * LANES, LANES)
            rowidx[r, sl] = rowidx[r, sl] + off
            return carry

        lax.fori_loop(0, nchp * vpc, shift_body, 0)
        plsc.subcore_barrier()

        def start_gather(u, j):
            pltpu.async_copy(g_hbm.at[rowidx.at[j]], rbuf[u], sem_g[u])

        def wait_gather(u):
            pltpu.make_async_copy(
                g_hbm.at[rowidx.at[0]], rbuf[u], sem_g[u]).wait()

        def start_scatter(u, j):
            pltpu.async_copy(rbuf[u], acc.at[colidx.at[j]], sem_s[u],
                             add=True)

        def wait_scatter(u):
            pltpu.make_async_copy(
                rbuf[u], acc.at[colidx.at[0]], sem_s[u]).wait()

        for u in range(PD):
            start_gather(u, u)

        def visit(u, j, prefetch, fresh_slot):
            wait_gather(u)
            start_scatter(u, j)
            if prefetch:
                up = (u + PD) % K
                if not fresh_slot:
                    wait_scatter(up)
                start_gather(up, j + PD)

        for u in range(K):
            visit(u, u, prefetch=True, fresh_slot=(u < K - PD))

        def mid_block(m, carry):
            j0 = m * K
            for u in range(K):
                visit(u, j0 + u, prefetch=True, fresh_slot=False)
            return carry

        lax.fori_loop(1, m_blocks - 1, mid_block, 0)

        jlast = (m_blocks - 1) * K
        for u in range(K):
            visit(u, jlast + u, prefetch=(u < K - PD), fresh_slot=False)
        for u in range(K):
            wait_scatter(u)
        plsc.subcore_barrier()

        pltpu.sync_copy(acc.at[pl.ds(base, rows_per_sub)],
                        out_hbm.at[pl.ds(base, rows_per_sub),
                                   pl.ds(cid * dh, dh)])

    return pl.kernel(
        body,
        out_type=jax.ShapeDtypeStruct((npad, 128), jnp.float32),
        mesh=mesh,
        compiler_params=pltpu.CompilerParams(use_tc_tiling_on_sc=False),
        scratch_types=(
            [pltpu.VMEM((nchp, CHUNK), jnp.int32)] * 2
            + [pltpu.VMEM((CHUNK, dh), jnp.float32)] * K
            + [pltpu.SemaphoreType.DMA] * (2 * K)
            + [pltpu.VMEM_SHARED((npad, dh), jnp.float32)]
        ),
    )(g_stack, rows3d, cols3d,
      jnp.zeros((rows_per_sub, dh), jnp.float32))


def _deg_to_dinv(d_ref, blk, n_valid, i):
    deg = d_ref[0, :, 0:1] + d_ref[1, :, 0:1]
    row = lax.broadcasted_iota(jnp.int32, (blk, 1), 0) + i * blk
    return jnp.where(row < n_valid,
                     lax.rsqrt(jnp.maximum(deg, 1.0)), 0.0)


def _tc1(x, w_fc, b_fc2d, deg128, n_valid, npad, blk=1024):
    _, d_in = x.shape
    d_h = w_fc.shape[0]

    def body(x_ref, w_ref, b_ref, d_ref, g_ref, dinv_ref):
        i = pl.program_id(0)
        dinv = _deg_to_dinv(d_ref, blk, n_valid, i)
        h = lax.dot_general(x_ref[...], w_ref[...],
                            (((1,), (1,)), ((), ())),
                            preferred_element_type=jnp.float32) + b_ref[...]
        g = jnp.where(dinv > 0.0, h * dinv, 0.0)
        dhh = d_h // 2
        g_ref[0] = g[:, :dhh]
        g_ref[1] = g[:, dhh:]
        dinv_ref[...] = dinv

    return pl.pallas_call(
        body,
        grid=(pl.cdiv(npad, blk),),
        in_specs=[
            pl.BlockSpec((blk, d_in), lambda i: (i, 0)),
            pl.BlockSpec((d_h, d_in), lambda i: (0, 0)),
            pl.BlockSpec((1, d_h), lambda i: (0, 0)),
            pl.BlockSpec((NC, blk, LANES), lambda i: (0, i, 0)),
        ],
        out_specs=[
            pl.BlockSpec((NC, blk, d_h // 2), lambda i: (0, i, 0)),
            pl.BlockSpec((blk, 1), lambda i: (i, 0)),
        ],
        out_shape=[
            jax.ShapeDtypeStruct((NC, npad, d_h // 2), jnp.float32),
            jax.ShapeDtypeStruct((npad, 1), jnp.float32),
        ],
    )(x, w_fc, b_fc2d, deg128)


def _tc2(s1, dinv2d, w2, d_h, blk=1024):
    npad = s1.shape[0]
    d_out = w2.shape[0]

    def body(s_ref, d_ref, w_ref, g_ref):
        dinv = d_ref[...]
        t = s_ref[:, :d_h] * dinv
        t = 1.0507009873554805 * jnp.where(
            t > 0.0, t, 1.6732632423543772 * (jnp.exp(t) - 1.0))
        h2 = lax.dot_general(t, w_ref[...], (((1,), (1,)), ((), ())),
                             preferred_element_type=jnp.float32)
        g2 = h2 * dinv
        doh = d_out // 2
        g_ref[0] = g2[:, :doh]
        g_ref[1] = g2[:, doh:]

    return pl.pallas_call(
        body,
        grid=(pl.cdiv(npad, blk),),
        in_specs=[
            pl.BlockSpec((blk, 128), lambda i: (i, 0)),
            pl.BlockSpec((blk, 1), lambda i: (i, 0)),
            pl.BlockSpec((d_out, d_h), lambda i: (0, 0)),
        ],
        out_specs=pl.BlockSpec((NC, blk, d_out // 2), lambda i: (0, i, 0)),
        out_shape=jax.ShapeDtypeStruct((NC, npad, d_out // 2), jnp.float32),
    )(s1, dinv2d, w2)


def _tc3(s2, dinv2d, b2_2d, n, d_out, blk=1000):
    def body(s_ref, d_ref, b_ref, o_ref):
        z = s_ref[:, :d_out] * d_ref[...] + b_ref[...]
        m = jnp.max(z, axis=1, keepdims=True)
        lse = jnp.log(jnp.sum(jnp.exp(z - m), axis=1, keepdims=True))
        o_ref[...] = z - m - lse

    return pl.pallas_call(
        body,
        grid=(n // blk,),
        in_specs=[
            pl.BlockSpec((blk, 128), lambda i: (i, 0)),
            pl.BlockSpec((blk, 1), lambda i: (i, 0)),
            pl.BlockSpec((1, d_out), lambda i: (0, 0)),
        ],
        out_specs=pl.BlockSpec((blk, d_out), lambda i: (i, 0)),
        out_shape=jax.ShapeDtypeStruct((n, d_out), jnp.float32),
    )(s2, dinv2d, b2_2d)


def kernel(x, edge_index, W_fc, b_fc, W2, b2):
    n, d_in = x.shape
    d_h = W_fc.shape[0]
    d_out = W2.shape[0]
    e = edge_index.shape[1]

    npad = _round_up(n + 8, CHUNK)
    total = e + n
    nchp = _round_up(-(-total // (NS * CHUNK)), 6)
    epad = NS * nchp * CHUNK
    nchd = nchp // NC

    loop = jnp.arange(n, dtype=jnp.int32)
    fill = jnp.full((epad - total,), n, jnp.int32)
    rows_flat = jnp.concatenate([edge_index[0], loop, fill])
    cols_flat = jnp.concatenate([edge_index[1], loop, fill])
    rows3d = rows_flat.reshape(NS, nchp, CHUNK)
    cols3d = cols_flat.reshape(NS, nchp, CHUNK)

    deg128 = _sc_degree(
        cols_flat.reshape(NW, nchd, CHUNK),
        jnp.ones((CHUNK, LANES), jnp.float32),
        jnp.zeros((npad // NS, LANES), jnp.float32),
        npad, nchd)
    g1, dinv = _tc1(x, W_fc, b_fc.reshape(1, -1), deg128, n, npad)
    s1 = _sc_propagate(g1.reshape(NC * npad, d_h // NC), rows3d, cols3d,
                       npad, nchp, d_h // NC)
    g2 = _tc2(s1, dinv, W2, d_h)
    s2 = _sc_propagate(g2.reshape(NC * npad, d_out // NC), rows3d, cols3d,
                       npad, nchp, d_out // NC)
    return _tc3(s2, dinv, b2.reshape(1, -1), n, d_out)


# speedup vs baseline: 1.0547x; 1.0184x over previous
import jax
import jax.numpy as jnp
from jax import lax
from jax.experimental import pallas as pl
from jax.experimental.pallas import tpu as pltpu
from jax.experimental.pallas import tpu_sc as plsc

NC = 2
NS = 16
LANES = 16
NW = NC * NS
CHUNK = 128


def _round_up(v, m):
    return (v + m - 1) // m * m


def _sc_degree(cols2d, ones_hbm, zeros_hbm, npad, nch):
    mesh = plsc.VectorSubcoreMesh(core_axis_name="c", subcore_axis_name="s")
    rows_per_sub = npad // NS

    def body(cols_hbm, ones_h, zeros_h, out_hbm, colidx, buf, acc, sem):
        cid = lax.axis_index("c")
        sid = lax.axis_index("s")
        wid = sid * NC + cid
        pltpu.sync_copy(cols_hbm.at[wid], colidx)
        pltpu.sync_copy(ones_h, buf)
        base = sid * rows_per_sub
        pltpu.sync_copy(zeros_h, acc.at[pl.ds(base, rows_per_sub)])
        plsc.subcore_barrier()

        def fire_body(j, carry):
            pltpu.async_copy(buf, acc.at[colidx.at[j]], sem, add=True)
            return carry

        lax.fori_loop(0, nch, fire_body, 0)

        def drain_body(j, carry):
            pltpu.make_async_copy(buf, acc.at[colidx.at[0]], sem).wait()
            return carry

        lax.fori_loop(0, nch, drain_body, 0)
        plsc.subcore_barrier()
        pltpu.sync_copy(acc.at[pl.ds(base, rows_per_sub)],
                        out_hbm.at[cid, pl.ds(base, rows_per_sub)])

    return pl.kernel(
        body,
        out_type=jax.ShapeDtypeStruct((NC, npad, LANES), jnp.float32),
        mesh=mesh,
        compiler_params=pltpu.CompilerParams(use_tc_tiling_on_sc=False),
        scratch_types=[
            pltpu.VMEM((nch, CHUNK), jnp.int32),
            pltpu.VMEM((CHUNK, LANES), jnp.float32),
            pltpu.VMEM_SHARED((npad, LANES), jnp.float32),
            pltpu.SemaphoreType.DMA,
        ],
    )(cols2d, ones_hbm, zeros_hbm)


def _sc_propagate(g_stack, rows4d, cols3d, npad, nchp, dh):
    mesh = plsc.VectorSubcoreMesh(core_axis_name="c", subcore_axis_name="s")
    rows_per_sub = npad // NS
    K = 6
    PD = 3
    m_blocks = nchp // K

    def body(g_hbm, rows_hbm, cols_hbm, zeros_hbm, out_hbm,
             rowidx, colidx, *rest):
        rbuf = rest[:K]
        sem_g = rest[K:2 * K]
        sem_s = rest[2 * K:3 * K]
        acc = rest[3 * K]
        cid = lax.axis_index("c")
        sid = lax.axis_index("s")
        pltpu.sync_copy(rows_hbm.at[cid, sid], rowidx)
        pltpu.sync_copy(cols_hbm.at[sid], colidx)
        base = sid * rows_per_sub
        pltpu.sync_copy(zeros_hbm, acc.at[pl.ds(base, rows_per_sub)])
        plsc.subcore_barrier()

        def start_gather(u, j):
            pltpu.async_copy(g_hbm.at[rowidx.at[j]], rbuf[u], sem_g[u])

        def wait_gather(u):
            pltpu.make_async_copy(
                g_hbm.at[rowidx.at[0]], rbuf[u], sem_g[u]).wait()

        def start_scatter(u, j):
            pltpu.async_copy(rbuf[u], acc.at[colidx.at[j]], sem_s[u],
                             add=True)

        def wait_scatter(u):
            pltpu.make_async_copy(
                rbuf[u], acc.at[colidx.at[0]], sem_s[u]).wait()

        for u in range(PD):
            start_gather(u, u)

        def visit(u, j, prefetch, fresh_slot):
            wait_gather(u)
            start_scatter(u, j)
            if prefetch:
                up = (u + PD) % K
                if not fresh_slot:
                    wait_scatter(up)
                start_gather(up, j + PD)

        for u in range(K):
            visit(u, u, prefetch=True, fresh_slot=(u < K - PD))

        def mid_block(m, carry):
            j0 = m * K
            for u in range(K):
                visit(u, j0 + u, prefetch=True, fresh_slot=False)
            return carry

        lax.fori_loop(1, m_blocks - 1, mid_block, 0)

        jlast = (m_blocks - 1) * K
        for u in range(K):
            visit(u, jlast + u, prefetch=(u < K - PD), fresh_slot=False)
        for u in range(K):
            wait_scatter(u)
        plsc.subcore_barrier()

        pltpu.sync_copy(acc.at[pl.ds(base, rows_per_sub)],
                        out_hbm.at[pl.ds(base, rows_per_sub),
                                   pl.ds(cid * dh, dh)])

    return pl.kernel(
        body,
        out_type=jax.ShapeDtypeStruct((npad, 128), jnp.float32),
        mesh=mesh,
        compiler_params=pltpu.CompilerParams(use_tc_tiling_on_sc=False),
        scratch_types=(
            [pltpu.VMEM((nchp, CHUNK), jnp.int32)] * 2
            + [pltpu.VMEM((CHUNK, dh), jnp.float32)] * K
            + [pltpu.SemaphoreType.DMA] * (2 * K)
            + [pltpu.VMEM_SHARED((npad, dh), jnp.float32)]
        ),
    )(g_stack, rows4d, cols3d,
      jnp.zeros((rows_per_sub, dh), jnp.float32))


def _deg_to_dinv(d_ref, blk, n_valid, i):
    deg = d_ref[0, :, 0:1] + d_ref[1, :, 0:1]
    row = lax.broadcasted_iota(jnp.int32, (blk, 1), 0) + i * blk
    return jnp.where(row < n_valid,
                     lax.rsqrt(jnp.maximum(deg, 1.0)), 0.0)


def _tc1(x, w_fc, b_fc2d, deg128, n_valid, npad, blk=1024):
    _, d_in = x.shape
    d_h = w_fc.shape[0]

    def body(x_ref, w_ref, b_ref, d_ref, g_ref, dinv_ref):
        i = pl.program_id(0)
        dinv = _deg_to_dinv(d_ref, blk, n_valid, i)
        h = lax.dot_general(x_ref[...], w_ref[...],
                            (((1,), (1,)), ((), ())),
                            preferred_element_type=jnp.float32) + b_ref[...]
        g = jnp.where(dinv > 0.0, h * dinv, 0.0)
        dhh = d_h // 2
        g_ref[0] = g[:, :dhh]
        g_ref[1] = g[:, dhh:]
        dinv_ref[...] = dinv

    return pl.pallas_call(
        body,
        grid=(pl.cdiv(npad, blk),),
        in_specs=[
            pl.BlockSpec((blk, d_in), lambda i: (i, 0)),
            pl.BlockSpec((d_h, d_in), lambda i: (0, 0)),
            pl.BlockSpec((1, d_h), lambda i: (0, 0)),
            pl.BlockSpec((NC, blk, LANES), lambda i: (0, i, 0)),
        ],
        out_specs=[
            pl.BlockSpec((NC, blk, d_h // 2), lambda i: (0, i, 0)),
            pl.BlockSpec((blk, 1), lambda i: (i, 0)),
        ],
        out_shape=[
            jax.ShapeDtypeStruct((NC, npad, d_h // 2), jnp.float32),
            jax.ShapeDtypeStruct((npad, 1), jnp.float32),
        ],
    )(x, w_fc, b_fc2d, deg128)


def _tc2(s1, dinv2d, w2, d_h, blk=1024):
    npad = s1.shape[0]
    d_out = w2.shape[0]

    def body(s_ref, d_ref, w_ref, g_ref):
        dinv = d_ref[...]
        t = s_ref[:, :d_h] * dinv
        t = 1.0507009873554805 * jnp.where(
            t > 0.0, t, 1.6732632423543772 * (jnp.exp(t) - 1.0))
        h2 = lax.dot_general(t, w_ref[...], (((1,), (1,)), ((), ())),
                             preferred_element_type=jnp.float32)
        g2 = h2 * dinv
        doh = d_out // 2
        g_ref[0] = g2[:, :doh]
        g_ref[1] = g2[:, doh:]

    return pl.pallas_call(
        body,
        grid=(pl.cdiv(npad, blk),),
        in_specs=[
            pl.BlockSpec((blk, 128), lambda i: (i, 0)),
            pl.BlockSpec((blk, 1), lambda i: (i, 0)),
            pl.BlockSpec((d_out, d_h), lambda i: (0, 0)),
        ],
        out_specs=pl.BlockSpec((NC, blk, d_out // 2), lambda i: (0, i, 0)),
        out_shape=jax.ShapeDtypeStruct((NC, npad, d_out // 2), jnp.float32),
    )(s1, dinv2d, w2)


def _tc3(s2, dinv2d, b2_2d, n, d_out, blk=1000):
    def body(s_ref, d_ref, b_ref, o_ref):
        z = s_ref[:, :d_out] * d_ref[...] + b_ref[...]
        m = jnp.max(z, axis=1, keepdims=True)
        lse = jnp.log(jnp.sum(jnp.exp(z - m), axis=1, keepdims=True))
        o_ref[...] = z - m - lse

    return pl.pallas_call(
        body,
        grid=(n // blk,),
        in_specs=[
            pl.BlockSpec((blk, 128), lambda i: (i, 0)),
            pl.BlockSpec((blk, 1), lambda i: (i, 0)),
            pl.BlockSpec((1, d_out), lambda i: (0, 0)),
        ],
        out_specs=pl.BlockSpec((blk, d_out), lambda i: (i, 0)),
        out_shape=jax.ShapeDtypeStruct((n, d_out), jnp.float32),
    )(s2, dinv2d, b2_2d)


def kernel(x, edge_index, W_fc, b_fc, W2, b2):
    n, d_in = x.shape
    d_h = W_fc.shape[0]
    d_out = W2.shape[0]
    e = edge_index.shape[1]

    npad = _round_up(n + 8, CHUNK)
    total = e + n
    nchp = _round_up(-(-total // (NS * CHUNK)), 6)
    epad = NS * nchp * CHUNK
    nchd = nchp // NC

    loop = jnp.arange(n, dtype=jnp.int32)
    fill = jnp.full((epad - total,), n, jnp.int32)
    rows_flat = jnp.concatenate([edge_index[0], loop, fill])
    cols_flat = jnp.concatenate([edge_index[1], loop, fill])
    rows3d = rows_flat.reshape(NS, nchp, CHUNK)
    rows4d = jnp.stack([rows3d, rows3d + npad])
    cols3d = cols_flat.reshape(NS, nchp, CHUNK)

    deg128 = _sc_degree(
        cols_flat.reshape(NW, nchd, CHUNK),
        jnp.ones((CHUNK, LANES), jnp.float32),
        jnp.zeros((npad // NS, LANES), jnp.float32),
        npad, nchd)
    g1, dinv = _tc1(x, W_fc, b_fc.reshape(1, -1), deg128, n, npad)
    s1 = _sc_propagate(g1.reshape(NC * npad, d_h // NC), rows4d, cols3d,
                       npad, nchp, d_h // NC)
    g2 = _tc2(s1, dinv, W2, d_h)
    s2 = _sc_propagate(g2.reshape(NC * npad, d_out // NC), rows4d, cols3d,
                       npad, nchp, d_out // NC)
    return _tc3(s2, dinv, b2.reshape(1, -1), n, d_out)


# speedup vs baseline: 1.1270x; 1.0686x over previous
import jax
import jax.numpy as jnp
from jax import lax
from jax.experimental import pallas as pl
from jax.experimental.pallas import tpu as pltpu
from jax.experimental.pallas import tpu_sc as plsc

NC = 2
NS = 16
LANES = 16
NW = NC * NS
CHUNK = 128


def _round_up(v, m):
    return (v + m - 1) // m * m


def _sc_degree(cols2d, ones_hbm, zeros_hbm, npad, nch):
    mesh = plsc.VectorSubcoreMesh(core_axis_name="c", subcore_axis_name="s")
    rows_per_sub = npad // NS

    def body(cols_hbm, ones_h, zeros_h, out_hbm, colidx, buf, acc, sem):
        cid = lax.axis_index("c")
        sid = lax.axis_index("s")
        wid = sid * NC + cid
        pltpu.sync_copy(cols_hbm.at[wid], colidx)
        pltpu.sync_copy(ones_h, buf)
        base = sid * rows_per_sub
        pltpu.sync_copy(zeros_h, acc.at[pl.ds(base, rows_per_sub)])
        plsc.subcore_barrier()

        def fire_body(j, carry):
            pltpu.async_copy(buf, acc.at[colidx.at[j]], sem, add=True)
            return carry

        lax.fori_loop(0, nch, fire_body, 0)

        def drain_body(j, carry):
            pltpu.make_async_copy(buf, acc.at[colidx.at[0]], sem).wait()
            return carry

        lax.fori_loop(0, nch, drain_body, 0)
        plsc.subcore_barrier()
        pltpu.sync_copy(acc.at[pl.ds(base, rows_per_sub)],
                        out_hbm.at[cid, pl.ds(base, rows_per_sub)])

    return pl.kernel(
        body,
        out_type=jax.ShapeDtypeStruct((NC, npad, LANES), jnp.float32),
        mesh=mesh,
        compiler_params=pltpu.CompilerParams(use_tc_tiling_on_sc=False),
        scratch_types=[
            pltpu.VMEM((nch, CHUNK), jnp.int32),
            pltpu.VMEM((CHUNK, LANES), jnp.float32),
            pltpu.VMEM_SHARED((npad, LANES), jnp.float32),
            pltpu.SemaphoreType.DMA,
        ],
    )(cols2d, ones_hbm, zeros_hbm)


def _sc_propagate(g_stack, rows4d, cols3d, npad, nchp, dh):
    mesh = plsc.VectorSubcoreMesh(core_axis_name="c", subcore_axis_name="s")
    rows_per_sub = npad // NS
    K = 6
    PD = 4
    m_blocks = nchp // K

    def body(g_hbm, rows_hbm, cols_hbm, zeros_hbm, out_hbm,
             rowidx, colidx, *rest):
        rbuf = rest[:K]
        sem_g = rest[K:2 * K]
        sem_s = rest[2 * K:3 * K]
        acc = rest[3 * K]
        cid = lax.axis_index("c")
        sid = lax.axis_index("s")
        pltpu.sync_copy(rows_hbm.at[cid, sid], rowidx)
        pltpu.sync_copy(cols_hbm.at[sid], colidx)
        base = sid * rows_per_sub
        pltpu.sync_copy(zeros_hbm, acc.at[pl.ds(base, rows_per_sub)])
        plsc.subcore_barrier()

        def start_gather(u, j):
            pltpu.async_copy(g_hbm.at[rowidx.at[j]], rbuf[u], sem_g[u])

        def wait_gather(u):
            pltpu.make_async_copy(
                g_hbm.at[rowidx.at[0]], rbuf[u], sem_g[u]).wait()

        def start_scatter(u, j):
            pltpu.async_copy(rbuf[u], acc.at[colidx.at[j]], sem_s[u],
                             add=True)

        def wait_scatter(u):
            pltpu.make_async_copy(
                rbuf[u], acc.at[colidx.at[0]], sem_s[u]).wait()

        for u in range(PD):
            start_gather(u, u)

        def visit(u, j, prefetch, fresh_slot):
            wait_gather(u)
            start_scatter(u, j)
            if prefetch:
                up = (u + PD) % K
                if not fresh_slot:
                    wait_scatter(up)
                start_gather(up, j + PD)

        for u in range(K):
            visit(u, u, prefetch=True, fresh_slot=(u < K - PD))

        def mid_block(m, carry):
            j0 = m * K
            for u in range(K):
                visit(u, j0 + u, prefetch=True, fresh_slot=False)
            return carry

        lax.fori_loop(1, m_blocks - 1, mid_block, 0)

        jlast = (m_blocks - 1) * K
        for u in range(K):
            visit(u, jlast + u, prefetch=(u < K - PD), fresh_slot=False)
        for u in range(K):
            wait_scatter(u)
        plsc.subcore_barrier()

        pltpu.sync_copy(acc.at[pl.ds(base, rows_per_sub)],
                        out_hbm.at[pl.ds(base, rows_per_sub),
                                   pl.ds(cid * dh, dh)])

    return pl.kernel(
        body,
        out_type=jax.ShapeDtypeStruct((npad, 128), jnp.float32),
        mesh=mesh,
        compiler_params=pltpu.CompilerParams(use_tc_tiling_on_sc=False),
        scratch_types=(
            [pltpu.VMEM((nchp, CHUNK), jnp.int32)] * 2
            + [pltpu.VMEM((CHUNK, dh), jnp.float32)] * K
            + [pltpu.SemaphoreType.DMA] * (2 * K)
            + [pltpu.VMEM_SHARED((npad, dh), jnp.float32)]
        ),
    )(g_stack, rows4d, cols3d,
      jnp.zeros((rows_per_sub, dh), jnp.float32))


def _deg_to_dinv(d_ref, blk, n_valid, i):
    deg = d_ref[0, :, 0:1] + d_ref[1, :, 0:1]
    row = lax.broadcasted_iota(jnp.int32, (blk, 1), 0) + i * blk
    return jnp.where(row < n_valid,
                     lax.rsqrt(jnp.maximum(deg, 1.0)), 0.0)


def _tc1(x, w_fc, b_fc2d, deg128, n_valid, npad, blk=1024):
    _, d_in = x.shape
    d_h = w_fc.shape[0]

    def body(x_ref, w_ref, b_ref, d_ref, g_ref, dinv_ref):
        i = pl.program_id(0)
        dinv = _deg_to_dinv(d_ref, blk, n_valid, i)
        h = lax.dot_general(x_ref[...], w_ref[...],
                            (((1,), (1,)), ((), ())),
                            preferred_element_type=jnp.float32) + b_ref[...]
        g = jnp.where(dinv > 0.0, h * dinv, 0.0)
        dhh = d_h // 2
        g_ref[0] = g[:, :dhh]
        g_ref[1] = g[:, dhh:]
        dinv_ref[...] = dinv

    return pl.pallas_call(
        body,
        grid=(pl.cdiv(npad, blk),),
        in_specs=[
            pl.BlockSpec((blk, d_in), lambda i: (i, 0)),
            pl.BlockSpec((d_h, d_in), lambda i: (0, 0)),
            pl.BlockSpec((1, d_h), lambda i: (0, 0)),
            pl.BlockSpec((NC, blk, LANES), lambda i: (0, i, 0)),
        ],
        out_specs=[
            pl.BlockSpec((NC, blk, d_h // 2), lambda i: (0, i, 0)),
            pl.BlockSpec((blk, 1), lambda i: (i, 0)),
        ],
        out_shape=[
            jax.ShapeDtypeStruct((NC, npad, d_h // 2), jnp.float32),
            jax.ShapeDtypeStruct((npad, 1), jnp.float32),
        ],
    )(x, w_fc, b_fc2d, deg128)


def _tc2(s1, dinv2d, w2, d_h, blk=1024):
    npad = s1.shape[0]
    d_out = w2.shape[0]

    def body(s_ref, d_ref, w_ref, g_ref):
        dinv = d_ref[...]
        t = s_ref[:, :d_h] * dinv
        t = 1.0507009873554805 * jnp.where(
            t > 0.0, t, 1.6732632423543772 * (jnp.exp(t) - 1.0))
        h2 = lax.dot_general(t, w_ref[...], (((1,), (1,)), ((), ())),
                             preferred_element_type=jnp.float32)
        g2 = h2 * dinv
        doh = d_out // 2
        g_ref[0] = g2[:, :doh]
        g_ref[1] = g2[:, doh:]

    return pl.pallas_call(
        body,
        grid=(pl.cdiv(npad, blk),),
        in_specs=[
            pl.BlockSpec((blk, 128), lambda i: (i, 0)),
            pl.BlockSpec((blk, 1), lambda i: (i, 0)),
            pl.BlockSpec((d_out, d_h), lambda i: (0, 0)),
        ],
        out_specs=pl.BlockSpec((NC, blk, d_out // 2), lambda i: (0, i, 0)),
        out_shape=jax.ShapeDtypeStruct((NC, npad, d_out // 2), jnp.float32),
    )(s1, dinv2d, w2)


def _tc3(s2, dinv2d, b2_2d, n, d_out, blk=1000):
    def body(s_ref, d_ref, b_ref, o_ref):
        z = s_ref[:, :d_out] * d_ref[...] + b_ref[...]
        m = jnp.max(z, axis=1, keepdims=True)
        lse = jnp.log(jnp.sum(jnp.exp(z - m), axis=1, keepdims=True))
        o_ref[...] = z - m - lse

    return pl.pallas_call(
        body,
        grid=(n // blk,),
        in_specs=[
            pl.BlockSpec((blk, 128), lambda i: (i, 0)),
            pl.BlockSpec((blk, 1), lambda i: (i, 0)),
            pl.BlockSpec((1, d_out), lambda i: (0, 0)),
        ],
        out_specs=pl.BlockSpec((blk, d_out), lambda i: (i, 0)),
        out_shape=jax.ShapeDtypeStruct((n, d_out), jnp.float32),
    )(s2, dinv2d, b2_2d)


def kernel(x, edge_index, W_fc, b_fc, W2, b2):
    n, d_in = x.shape
    d_h = W_fc.shape[0]
    d_out = W2.shape[0]
    e = edge_index.shape[1]

    npad = _round_up(n + 8, CHUNK)
    total = e + n
    nchp = _round_up(-(-total // (NS * CHUNK)), 6)
    epad = NS * nchp * CHUNK
    nchd = nchp // NC

    loop = jnp.arange(n, dtype=jnp.int32)
    fill = jnp.full((epad - total,), n, jnp.int32)
    rows_flat = jnp.concatenate([edge_index[0], loop, fill])
    cols_flat = jnp.concatenate([edge_index[1], loop, fill])
    rows3d = rows_flat.reshape(NS, nchp, CHUNK)
    rows4d = jnp.stack([rows3d, rows3d + npad])
    cols3d = cols_flat.reshape(NS, nchp, CHUNK)

    deg128 = _sc_degree(
        cols_flat.reshape(NW, nchd, CHUNK),
        jnp.ones((CHUNK, LANES), jnp.float32),
        jnp.zeros((npad // NS, LANES), jnp.float32),
        npad, nchd)
    g1, dinv = _tc1(x, W_fc, b_fc.reshape(1, -1), deg128, n, npad)
    s1 = _sc_propagate(g1.reshape(NC * npad, d_h // NC), rows4d, cols3d,
                       npad, nchp, d_h // NC)
    g2 = _tc2(s1, dinv, W2, d_h)
    s2 = _sc_propagate(g2.reshape(NC * npad, d_out // NC), rows4d, cols3d,
                       npad, nchp, d_out // NC)
    return _tc3(s2, dinv, b2.reshape(1, -1), n, d_out)


# speedup vs baseline: 1.1324x; 1.0047x over previous
import jax
import jax.numpy as jnp
from jax import lax
from jax.experimental import pallas as pl
from jax.experimental.pallas import tpu as pltpu
from jax.experimental.pallas import tpu_sc as plsc

NC = 2
NS = 16
LANES = 16
NW = NC * NS
CHUNK = 128


def _round_up(v, m):
    return (v + m - 1) // m * m


def _sc_degree(cols2d, ones_hbm, zeros_hbm, npad, nch):
    mesh = plsc.VectorSubcoreMesh(core_axis_name="c", subcore_axis_name="s")
    rows_per_sub = npad // NS

    def body(cols_hbm, ones_h, zeros_h, out_hbm, colidx, buf, acc, sem):
        cid = lax.axis_index("c")
        sid = lax.axis_index("s")
        wid = sid * NC + cid
        pltpu.sync_copy(cols_hbm.at[wid], colidx)
        pltpu.sync_copy(ones_h, buf)
        base = sid * rows_per_sub
        pltpu.sync_copy(zeros_h, acc.at[pl.ds(base, rows_per_sub)])
        plsc.subcore_barrier()

        def fire_body(j, carry):
            pltpu.async_copy(buf, acc.at[colidx.at[j]], sem, add=True)
            return carry

        lax.fori_loop(0, nch, fire_body, 0)

        def drain_body(j, carry):
            pltpu.make_async_copy(buf, acc.at[colidx.at[0]], sem).wait()
            return carry

        lax.fori_loop(0, nch, drain_body, 0)
        plsc.subcore_barrier()
        pltpu.sync_copy(acc.at[pl.ds(base, rows_per_sub)],
                        out_hbm.at[cid, pl.ds(base, rows_per_sub)])

    return pl.kernel(
        body,
        out_type=jax.ShapeDtypeStruct((NC, npad, LANES), jnp.float32),
        mesh=mesh,
        compiler_params=pltpu.CompilerParams(use_tc_tiling_on_sc=False),
        scratch_types=[
            pltpu.VMEM((nch, CHUNK), jnp.int32),
            pltpu.VMEM((CHUNK, LANES), jnp.float32),
            pltpu.VMEM_SHARED((npad, LANES), jnp.float32),
            pltpu.SemaphoreType.DMA,
        ],
    )(cols2d, ones_hbm, zeros_hbm)


def _sc_propagate(g_stack, rows4d, cols3d, npad, nchp, dh):
    mesh = plsc.VectorSubcoreMesh(core_axis_name="c", subcore_axis_name="s")
    rows_per_sub = npad // NS
    K = 6
    PD = 5
    m_blocks = nchp // K

    def body(g_hbm, rows_hbm, cols_hbm, zeros_hbm, out_hbm,
             rowidx, colidx, *rest):
        rbuf = rest[:K]
        sem_g = rest[K:2 * K]
        sem_s = rest[2 * K:3 * K]
        acc = rest[3 * K]
        cid = lax.axis_index("c")
        sid = lax.axis_index("s")
        pltpu.sync_copy(rows_hbm.at[cid, sid], rowidx)
        pltpu.sync_copy(cols_hbm.at[sid], colidx)
        base = sid * rows_per_sub
        pltpu.sync_copy(zeros_hbm, acc.at[pl.ds(base, rows_per_sub)])
        plsc.subcore_barrier()

        def start_gather(u, j):
            pltpu.async_copy(g_hbm.at[rowidx.at[j]], rbuf[u], sem_g[u])

        def wait_gather(u):
            pltpu.make_async_copy(
                g_hbm.at[rowidx.at[0]], rbuf[u], sem_g[u]).wait()

        def start_scatter(u, j):
            pltpu.async_copy(rbuf[u], acc.at[colidx.at[j]], sem_s[u],
                             add=True)

        def wait_scatter(u):
            pltpu.make_async_copy(
                rbuf[u], acc.at[colidx.at[0]], sem_s[u]).wait()

        for u in range(PD):
            start_gather(u, u)

        def visit(u, j, prefetch, fresh_slot):
            wait_gather(u)
            start_scatter(u, j)
            if prefetch:
                up = (u + PD) % K
                if not fresh_slot:
                    wait_scatter(up)
                start_gather(up, j + PD)

        for u in range(K):
            visit(u, u, prefetch=True, fresh_slot=(u < K - PD))

        def mid_block(m, carry):
            j0 = m * K
            for u in range(K):
                visit(u, j0 + u, prefetch=True, fresh_slot=False)
            return carry

        lax.fori_loop(1, m_blocks - 1, mid_block, 0)

        jlast = (m_blocks - 1) * K
        for u in range(K):
            visit(u, jlast + u, prefetch=(u < K - PD), fresh_slot=False)
        for u in range(K):
            wait_scatter(u)
        plsc.subcore_barrier()

        pltpu.sync_copy(acc.at[pl.ds(base, rows_per_sub)],
                        out_hbm.at[pl.ds(base, rows_per_sub),
                                   pl.ds(cid * dh, dh)])

    return pl.kernel(
        body,
        out_type=jax.ShapeDtypeStruct((npad, 128), jnp.float32),
        mesh=mesh,
        compiler_params=pltpu.CompilerParams(use_tc_tiling_on_sc=False),
        scratch_types=(
            [pltpu.VMEM((nchp, CHUNK), jnp.int32)] * 2
            + [pltpu.VMEM((CHUNK, dh), jnp.float32)] * K
            + [pltpu.SemaphoreType.DMA] * (2 * K)
            + [pltpu.VMEM_SHARED((npad, dh), jnp.float32)]
        ),
    )(g_stack, rows4d, cols3d,
      jnp.zeros((rows_per_sub, dh), jnp.float32))


def _deg_to_dinv(d_ref, blk, n_valid, i):
    deg = d_ref[0, :, 0:1] + d_ref[1, :, 0:1]
    row = lax.broadcasted_iota(jnp.int32, (blk, 1), 0) + i * blk
    return jnp.where(row < n_valid,
                     lax.rsqrt(jnp.maximum(deg, 1.0)), 0.0)


def _tc1(x, w_fc, b_fc2d, deg128, n_valid, npad, blk=1024):
    _, d_in = x.shape
    d_h = w_fc.shape[0]

    def body(x_ref, w_ref, b_ref, d_ref, g_ref, dinv_ref):
        i = pl.program_id(0)
        dinv = _deg_to_dinv(d_ref, blk, n_valid, i)
        h = lax.dot_general(x_ref[...], w_ref[...],
                            (((1,), (1,)), ((), ())),
                            preferred_element_type=jnp.float32) + b_ref[...]
        g = jnp.where(dinv > 0.0, h * dinv, 0.0)
        dhh = d_h // 2
        g_ref[0] = g[:, :dhh]
        g_ref[1] = g[:, dhh:]
        dinv_ref[...] = dinv

    return pl.pallas_call(
        body,
        grid=(pl.cdiv(npad, blk),),
        in_specs=[
            pl.BlockSpec((blk, d_in), lambda i: (i, 0)),
            pl.BlockSpec((d_h, d_in), lambda i: (0, 0)),
            pl.BlockSpec((1, d_h), lambda i: (0, 0)),
            pl.BlockSpec((NC, blk, LANES), lambda i: (0, i, 0)),
        ],
        out_specs=[
            pl.BlockSpec((NC, blk, d_h // 2), lambda i: (0, i, 0)),
            pl.BlockSpec((blk, 1), lambda i: (i, 0)),
        ],
        out_shape=[
            jax.ShapeDtypeStruct((NC, npad, d_h // 2), jnp.float32),
            jax.ShapeDtypeStruct((npad, 1), jnp.float32),
        ],
    )(x, w_fc, b_fc2d, deg128)


def _tc2(s1, dinv2d, w2, d_h, blk=1024):
    npad = s1.shape[0]
    d_out = w2.shape[0]

    def body(s_ref, d_ref, w_ref, g_ref):
        dinv = d_ref[...]
        t = s_ref[:, :d_h] * dinv
        t = 1.0507009873554805 * jnp.where(
            t > 0.0, t, 1.6732632423543772 * (jnp.exp(t) - 1.0))
        h2 = lax.dot_general(t, w_ref[...], (((1,), (1,)), ((), ())),
                             preferred_element_type=jnp.float32)
        g2 = h2 * dinv
        doh = d_out // 2
        g_ref[0] = g2[:, :doh]
        g_ref[1] = g2[:, doh:]

    return pl.pallas_call(
        body,
        grid=(pl.cdiv(npad, blk),),
        in_specs=[
            pl.BlockSpec((blk, 128), lambda i: (i, 0)),
            pl.BlockSpec((blk, 1), lambda i: (i, 0)),
            pl.BlockSpec((d_out, d_h), lambda i: (0, 0)),
        ],
        out_specs=pl.BlockSpec((NC, blk, d_out // 2), lambda i: (0, i, 0)),
        out_shape=jax.ShapeDtypeStruct((NC, npad, d_out // 2), jnp.float32),
    )(s1, dinv2d, w2)


def _tc3(s2, dinv2d, b2_2d, n, d_out, blk=1000):
    def body(s_ref, d_ref, b_ref, o_ref):
        z = s_ref[:, :d_out] * d_ref[...] + b_ref[...]
        m = jnp.max(z, axis=1, keepdims=True)
        lse = jnp.log(jnp.sum(jnp.exp(z - m), axis=1, keepdims=True))
        o_ref[...] = z - m - lse

    return pl.pallas_call(
        body,
        grid=(n // blk,),
        in_specs=[
            pl.BlockSpec((blk, 128), lambda i: (i, 0)),
            pl.BlockSpec((blk, 1), lambda i: (i, 0)),
            pl.BlockSpec((1, d_out), lambda i: (0, 0)),
        ],
        out_specs=pl.BlockSpec((blk, d_out), lambda i: (i, 0)),
        out_shape=jax.ShapeDtypeStruct((n, d_out), jnp.float32),
    )(s2, dinv2d, b2_2d)


def kernel(x, edge_index, W_fc, b_fc, W2, b2):
    n, d_in = x.shape
    d_h = W_fc.shape[0]
    d_out = W2.shape[0]
    e = edge_index.shape[1]

    npad = _round_up(n + 8, CHUNK)
    total = e + n
    nchp = _round_up(-(-total // (NS * CHUNK)), 6)
    epad = NS * nchp * CHUNK
    nchd = nchp // NC

    loop = jnp.arange(n, dtype=jnp.int32)
    fill = jnp.full((epad - total,), n, jnp.int32)
    rows_flat = jnp.concatenate([edge_index[0], loop, fill])
    cols_flat = jnp.concatenate([edge_index[1], loop, fill])
    rows3d = rows_flat.reshape(NS, nchp, CHUNK)
    rows4d = jnp.stack([rows3d, rows3d + npad])
    cols3d = cols_flat.reshape(NS, nchp, CHUNK)

    deg128 = _sc_degree(
        cols_flat.reshape(NW, nchd, CHUNK),
        jnp.ones((CHUNK, LANES), jnp.float32),
        jnp.zeros((npad // NS, LANES), jnp.float32),
        npad, nchd)
    g1, dinv = _tc1(x, W_fc, b_fc.reshape(1, -1), deg128, n, npad)
    s1 = _sc_propagate(g1.reshape(NC * npad, d_h // NC), rows4d, cols3d,
                       npad, nchp, d_h // NC)
    g2 = _tc2(s1, dinv, W2, d_h)
    s2 = _sc_propagate(g2.reshape(NC * npad, d_out // NC), rows4d, cols3d,
                       npad, nchp, d_out // NC)
    return _tc3(s2, dinv, b2.reshape(1, -1), n, d_out)
